# Initial kernel scaffold; baseline (speedup 1.0000x reference)
import jax
import jax.numpy as jnp
from jax.experimental import pallas as pl


def kernel(rel_pos_emb_k, rel_pos_emb_v, L):
    raise NotImplementedError("write your pallas kernel here")


# speedup vs baseline: 2.6630x; 2.6630x over previous
import functools

import jax
import jax.numpy as jnp
from jax import lax
from jax.experimental import pallas as pl
from jax.experimental.pallas import tpu as pltpu
from jax.experimental.pallas import tpu_sc as plsc

MAX_REL = 128
NUM_HEADS = 8
D_K = 64
L_STATIC = 512
D = NUM_HEADS * D_K
F_ROWS = 1024
NC = 2
NS = 16
NW = NC * NS
ROWS_PER_SUB = F_ROWS // NS
I_PER_W = L_STATIC // NW
OUT_ROW = L_STATIC * D


def _sc_body(tk_hbm, tv_hbm, outk_hbm, outv_hbm, fk_sh, fv_sh, sem):
    cid = lax.axis_index("c")
    sid = lax.axis_index("s")

    base = sid * ROWS_PER_SUB
    fills = []
    for r in range(ROWS_PER_SUB):
        s = base + r
        tidx = jnp.clip(511 - s, -MAX_REL, MAX_REL) + MAX_REL
        src = pl.multiple_of(tidx * D, D)
        dst = pl.multiple_of(s * D, D)
        fills.append(
            pltpu.async_copy(tk_hbm.at[pl.ds(src, D)], fk_sh.at[pl.ds(dst, D)], sem))
        fills.append(
            pltpu.async_copy(tv_hbm.at[pl.ds(src, D)], fv_sh.at[pl.ds(dst, D)], sem))
    for c in fills:
        c.wait()
    plsc.subcore_barrier()

    wid = sid * NC + cid
    copies = []
    for t in range(I_PER_W):
        i = wid * I_PER_W + t
        src = pl.multiple_of((511 - i) * D, D)
        dst = pl.multiple_of(i * OUT_ROW, D)
        copies.append(
            pltpu.async_copy(fk_sh.at[pl.ds(src, OUT_ROW)],
                             outk_hbm.at[pl.ds(dst, OUT_ROW)], sem))
        copies.append(
            pltpu.async_copy(fv_sh.at[pl.ds(src, OUT_ROW)],
                             outv_hbm.at[pl.ds(dst, OUT_ROW)], sem))
    for c in copies:
        c.wait()


@jax.jit
def _rel_emb(tk, tv):
    mesh = plsc.VectorSubcoreMesh(core_axis_name="c", subcore_axis_name="s")
    f = functools.partial(
        pl.kernel,
        mesh=mesh,
        out_type=(
            jax.ShapeDtypeStruct((L_STATIC * OUT_ROW,), jnp.float32),
            jax.ShapeDtypeStruct((L_STATIC * OUT_ROW,), jnp.float32),
        ),
        scratch_types=[
            pltpu.VMEM_SHARED((F_ROWS * D,), jnp.float32),
            pltpu.VMEM_SHARED((F_ROWS * D,), jnp.float32),
            pltpu.SemaphoreType.DMA,
        ],
    )(_sc_body)
    return f(tk, tv)


def kernel(rel_pos_emb_k, rel_pos_emb_v, L):
    tk = rel_pos_emb_k.reshape(-1)
    tv = rel_pos_emb_v.reshape(-1)
    outk, outv = _rel_emb(tk, tv)
    outk = outk.reshape(L_STATIC, L_STATIC, NUM_HEADS, D_K)
    outv = outv.reshape(L_STATIC, L_STATIC, NUM_HEADS, D_K)
    return (outk, outv)
